# Initial kernel scaffold; baseline (speedup 1.0000x reference)
import jax
import jax.numpy as jnp
from jax.experimental import pallas as pl


def kernel(x, edge_index, edge_attr, params):
    raise NotImplementedError("write your pallas kernel here")


# speedup vs baseline: 2.3910x; 2.3910x over previous
import functools

import jax
import jax.numpy as jnp
from jax import lax
from jax.experimental import pallas as pl
from jax.experimental.pallas import tpu as pltpu
from jax.experimental.pallas import tpu_sc as plsc

N_NODES = 10000
N_EDGES = 320000
F = 80
VF = 48
D = F + VF

NC, NS = 2, 16
NW = NC * NS
EPW = N_EDGES // NW
K = 80
NCHUNK = EPW // K
RPT = 624
REM = N_NODES - RPT * NS

def _gather_body(x_hbm, src_hbm, dst_hbm, xj_hbm, outc_hbm,
                 accc, idx_v, rows_v, didx_v, ones_v, stg_v, sem):
    cid = lax.axis_index("c")
    sid = lax.axis_index("s")
    wid = sid * NC + cid
    base = wid * EPW
    rows0 = pl.multiple_of(sid * RPT, 8)

    _fill16(ones_v, 1.0)
    _fill16(stg_v, 0.0)

    def zstep(t, carry):
        r = pl.multiple_of(rows0 + t * ZCH, 8)
        pltpu.sync_copy(stg_v, accc.at[pl.ds(r, ZCH)])
        return carry

    lax.fori_loop(0, RPT // ZCH, zstep, 0)

    @pl.when(sid == NS - 1)
    def _():
        pltpu.sync_copy(stg_v.at[pl.ds(0, REM)],
                        accc.at[pl.ds(RPT * NS, REM)])

    plsc.subcore_barrier()

    def step(j, carry):
        off = pl.multiple_of(base + j * K, K)
        pltpu.sync_copy(src_hbm.at[pl.ds(off, K)], idx_v)
        pltpu.async_copy(x_hbm.at[idx_v], rows_v, sem).wait()
        pltpu.sync_copy(rows_v, xj_hbm.at[pl.ds(off, K)])
        pltpu.sync_copy(dst_hbm.at[pl.ds(off, K)], didx_v)
        pltpu.sync_copy(ones_v, accc.at[didx_v], add=True)
        return carry

    lax.fori_loop(0, NCHUNK, step, 0)
    plsc.subcore_barrier()

    def ostep(t, carry):
        r = pl.multiple_of(rows0 + t * ZCH, 8)
        pltpu.sync_copy(accc.at[pl.ds(r, ZCH)], stg_v)
        pltpu.sync_copy(stg_v, outc_hbm.at[cid, pl.ds(r, ZCH)])
        return carry

    lax.fori_loop(0, RPT // ZCH, ostep, 0)

    @pl.when(sid == NS - 1)
    def _():
        pltpu.sync_copy(accc.at[pl.ds(RPT * NS, REM)], stg_v.at[pl.ds(0, REM)])
        pltpu.sync_copy(stg_v.at[pl.ds(0, REM)],
                        outc_hbm.at[cid, pl.ds(RPT * NS, REM)])


@functools.cache
def _gather():
    mesh = plsc.VectorSubcoreMesh(
        core_axis_name="c", subcore_axis_name="s",
        num_cores=NC, num_subcores=NS)
    return pl.kernel(
        _gather_body,
        out_type=(jax.ShapeDtypeStruct((N_EDGES, D), jnp.float32),
                  jax.ShapeDtypeStruct((NC, N_NODES, D), jnp.float32)),
        mesh=mesh,
        scratch_types=[
            pltpu.VMEM_SHARED((N_NODES, D), jnp.float32),
            pltpu.VMEM((K,), jnp.int32),
            pltpu.VMEM((K, D), jnp.float32),
            pltpu.VMEM((K,), jnp.int32),
            pltpu.VMEM((K, D), jnp.float32),
            pltpu.VMEM((ZCH, D), jnp.float32),
            pltpu.SemaphoreType.DMA,
        ],
    )


ZCH = 48


def _fill16(ref, val):
    rows, cols = ref.shape
    cpr = cols // 16
    vec = jnp.full((16,), val, jnp.float32)

    def body(i, carry):
        ref[i // cpr, pl.ds((i % cpr) * 16, 16)] = vec
        return carry

    lax.fori_loop(0, rows * cpr, body, 0)


def _scatter_body(msg_hbm, dst_hbm, outm_hbm, accm, idx_v, msg_v, stgm_v):
    cid = lax.axis_index("c")
    sid = lax.axis_index("s")
    wid = sid * NC + cid
    rows0 = pl.multiple_of(sid * RPT, 8)

    _fill16(stgm_v, 0.0)

    def zstep(t, carry):
        r = pl.multiple_of(rows0 + t * ZCH, 8)
        pltpu.sync_copy(stgm_v, accm.at[pl.ds(r, ZCH)])
        return carry

    lax.fori_loop(0, RPT // ZCH, zstep, 0)

    @pl.when(sid == NS - 1)
    def _():
        pltpu.sync_copy(stgm_v.at[pl.ds(0, REM)],
                        accm.at[pl.ds(RPT * NS, REM)])

    plsc.subcore_barrier()

    def step(j, carry):
        off = pl.multiple_of(wid * EPW + j * K, K)
        pltpu.sync_copy(dst_hbm.at[pl.ds(off, K)], idx_v)
        pltpu.sync_copy(msg_hbm.at[pl.ds(off, K)], msg_v)
        pltpu.sync_copy(msg_v, accm.at[idx_v], add=True)
        return carry

    lax.fori_loop(0, NCHUNK, step, 0)
    plsc.subcore_barrier()

    def ostep(t, carry):
        r = pl.multiple_of(rows0 + t * ZCH, 8)
        pltpu.sync_copy(accm.at[pl.ds(r, ZCH)], stgm_v)
        pltpu.sync_copy(stgm_v, outm_hbm.at[cid, pl.ds(r, ZCH)])
        return carry

    lax.fori_loop(0, RPT // ZCH, ostep, 0)

    @pl.when(sid == NS - 1)
    def _():
        pltpu.sync_copy(accm.at[pl.ds(RPT * NS, REM)], stgm_v.at[pl.ds(0, REM)])
        pltpu.sync_copy(stgm_v.at[pl.ds(0, REM)],
                        outm_hbm.at[cid, pl.ds(RPT * NS, REM)])


@functools.cache
def _scatter():
    mesh = plsc.VectorSubcoreMesh(
        core_axis_name="c", subcore_axis_name="s",
        num_cores=NC, num_subcores=NS)
    return pl.kernel(
        _scatter_body,
        out_type=jax.ShapeDtypeStruct((NC, N_NODES, D), jnp.float32),
        mesh=mesh,
        scratch_types=[
            pltpu.VMEM_SHARED((N_NODES, D), jnp.float32),
            pltpu.VMEM((K,), jnp.int32),
            pltpu.VMEM((K, D), jnp.float32),
            pltpu.VMEM((ZCH, D), jnp.float32),
        ],
    )


def _dot(a, b):
    return jnp.dot(a, b, preferred_element_type=jnp.float32)


def _gvp(f, v, WhE, WuE, Ssh, Sg, SgT, Wff, Wfs, b):
    Vh = _dot(v, WhE)
    sh = jnp.sqrt(_dot(Vh * Vh, Ssh) + 1e-12)
    Vu = _dot(Vh, WuE)
    fo = jax.nn.sigmoid(_dot(f, Wff) + _dot(sh, Wfs) + b)
    g = jnp.sqrt(_dot(Vu * Vu, Sg) + 1e-12)
    vo = _dot(jax.nn.sigmoid(g), SgT) * Vu
    return fo, vo


def _edge_body(xj, ea,
               WhEx, WhEe, WuE1, S32, S16, S16T,
               Wf1x, Wf1e, Wf1s, b1,
               WhE2, WuE2, Wf2f, Wf2s, b2,
               WhE3, WuE3, Wf3f, Wf3s, b3,
               out):
    xjv = xj[:, F:]
    eav = ea[:, F:]
    Vh = _dot(xjv, WhEx[...]) + _dot(eav, WhEe[...])
    sh = jnp.sqrt(_dot(Vh * Vh, S32[...]) + 1e-12)
    Vu = _dot(Vh, WuE1[...])
    f = jax.nn.sigmoid(_dot(xj[:, :F], Wf1x[...]) + _dot(ea[:, :F], Wf1e[...])
                       + _dot(sh, Wf1s[...]) + b1[...])
    g = jnp.sqrt(_dot(Vu * Vu, S16[...]) + 1e-12)
    v = _dot(jax.nn.sigmoid(g), S16T[...]) * Vu
    f, v = _gvp(f, v, WhE2[...], WuE2[...], S16[...], S16[...], S16T[...],
                Wf2f[...], Wf2s[...], b2[...])
    f, v = _gvp(f, v, WhE3[...], WuE3[...], S16[...], S16[...], S16T[...],
                Wf3f[...], Wf3s[...], b3[...])
    out[:, :F] = f
    out[:, F:] = v


def _node_body(pm, pc,
               ln0g, ln0b,
               WhEa, WuEa, S32n, S32nT, WfAf, WfAs, bA,
               WhEb, WuEb, S16n, S16nT, WfBf, WfBs, bB,
               ln1g, ln1b, Wdf, Wdv, bd,
               out):
    agg = pm[0] + pm[1]
    cnt = pc[0, :, :1] + pc[1, :, :1]
    inv = 1.0 / jnp.maximum(cnt, 1.0)
    f = agg[:, :F] * inv
    vf = agg[:, F:] * inv
    mu = jnp.mean(f, axis=1, keepdims=True)
    var = jnp.mean((f - mu) ** 2, axis=1, keepdims=True)
    nf = (f - mu) / jnp.sqrt(var + 1e-5) * ln0g[...] + ln0b[...]
    vn = jnp.sqrt(jnp.sum(vf * vf, axis=1, keepdims=True))
    vln = vf / (vn + 1e-8)
    f2, v2 = _gvp(nf, vln, WhEa[...], WuEa[...], S32n[...], S32n[...],
                  S32nT[...], WfAf[...], WfAs[...], bA[...])
    f2, v2 = _gvp(f2, v2, WhEb[...], WuEb[...], S32n[...], S16n[...],
                  S16nT[...], WfBf[...], WfBs[...], bB[...])
    f3 = nf + f2
    v3 = vln + v2
    mu2 = jnp.mean(f3, axis=1, keepdims=True)
    var2 = jnp.mean((f3 - mu2) ** 2, axis=1, keepdims=True)
    nf2 = (f3 - mu2) / jnp.sqrt(var2 + 1e-5) * ln1g[...] + ln1b[...]
    vn2 = jnp.sqrt(jnp.sum(v3 * v3, axis=1, keepdims=True))
    v3n = v3 / (vn2 + 1e-8)
    out[...] = _dot(nf2, Wdf[...]) + _dot(v3n, Wdv[...]) + bd[...]


BE = 2000
BN = 1000


def _full(a):
    return pl.BlockSpec(a.shape, lambda i: tuple(0 for _ in a.shape))


def _edge_call(xj, ea, weights):
    specs = ([pl.BlockSpec((BE, D), lambda i: (i, 0))] * 2
             + [_full(w) for w in weights])
    return pl.pallas_call(
        _edge_body,
        grid=(N_EDGES // BE,),
        in_specs=specs,
        out_specs=pl.BlockSpec((BE, D), lambda i: (i, 0)),
        out_shape=jax.ShapeDtypeStruct((N_EDGES, D), jnp.float32),
    )(xj, ea, *weights)


def _node_call(pm, pc, weights):
    specs = ([pl.BlockSpec((NC, BN, D), lambda i: (0, i, 0)),
              pl.BlockSpec((NC, BN, D), lambda i: (0, i, 0))]
             + [_full(w) for w in weights])
    return pl.pallas_call(
        _node_body,
        grid=(N_NODES // BN,),
        in_specs=specs,
        out_specs=pl.BlockSpec((BN, 1), lambda i: (i, 0)),
        out_shape=jax.ShapeDtypeStruct((N_NODES, 1), jnp.float32),
    )(pm, pc, *weights)


def _expand(W):
    return jnp.kron(W, jnp.eye(3, dtype=W.dtype))


def _sel(h):
    return jnp.kron(jnp.eye(h, dtype=jnp.float32), jnp.ones((3, 1), jnp.float32))


def _edge_weights(lp):
    g1, g2, g3 = lp['wev']
    WhE1 = _expand(g1['Wh'])
    S32 = _sel(32)
    S16 = _sel(16)
    return (WhE1[:VF], WhE1[VF:], _expand(g1['Wu']), S32, S16, S16.T,
            g1['Wf'][:F], g1['Wf'][F:2 * F], g1['Wf'][2 * F:],
            g1['bf'][None, :],
            _expand(g2['Wh']), _expand(g2['Wu']),
            g2['Wf'][:F], g2['Wf'][F:], g2['bf'][None, :],
            _expand(g3['Wh']), _expand(g3['Wu']),
            g3['Wf'][:F], g3['Wf'][F:], g3['bf'][None, :])


def _node_weights(lp, Wd, bd):
    ga, gb = lp['wdh']
    S32 = _sel(32)
    S16 = _sel(16)
    return (lp['ln0_g'][None, :], lp['ln0_b'][None, :],
            _expand(ga['Wh']), _expand(ga['Wu']), S32, S32.T,
            ga['Wf'][:F], ga['Wf'][F:], ga['bf'][None, :],
            _expand(gb['Wh']), _expand(gb['Wu']), S16, S16.T,
            gb['Wf'][:4 * F], gb['Wf'][4 * F:], gb['bf'][None, :],
            lp['ln1_g'][None, :], lp['ln1_b'][None, :],
            Wd[:F], Wd[F:], bd[None, :])


def kernel(x, edge_index, edge_attr, params):
    lp = params['layers'][-1]
    src, dst = edge_index[0], edge_index[1]
    xj, pc = _gather()(x, src, dst)
    msg = _edge_call(xj, edge_attr, _edge_weights(lp))
    pm = _scatter()(msg, dst)
    return _node_call(pm, pc, _node_weights(lp, params['Wd'], params['bd']))
